# baseline (speedup 1.0000x reference)
import functools

import jax
import jax.numpy as jnp
from jax import lax
from jax.experimental import pallas as pl
from jax.experimental.pallas import tpu as pltpu
from jax.experimental.pallas import tpu_sc as plsc

B, N, NBONDS = 20, 1000, 1000
MAX_NB = 10
H = 128
ATOM_FDIM = 82
BOND_FDIM = 6

NR = 20480
PADROW = B * N
NEG = -1e30

NW = 32
ROWS_PER_W = NR // NW
CH_ROWS = 32
N_CH = ROWS_PER_W // CH_ROWS
SUB = 80
NSUB = CH_ROWS * MAX_NB // SUB
EROWS = NR * MAX_NB // SUB

_f32 = jnp.float32


def _valid_rows(i, shape):
    rows = i * shape[0] + lax.broadcasted_iota(jnp.int32, shape, 0)
    return rows < PADROW


def _prep_body(ag_ref, bg_ref, nn_ref, ago_ref, bgo_ref):
    b = pl.program_id(0)
    k_idx = lax.broadcasted_iota(jnp.int32, (1, N, MAX_NB), 2)
    valid = k_idx < nn_ref[...]
    ago_ref[...] = jnp.where(valid, ag_ref[...] + b * N, PADROW)
    bgo_ref[...] = jnp.where(valid, bg_ref[...] + b * NBONDS, PADROW)


_prep = pl.pallas_call(
    _prep_body,
    grid=(B,),
    in_specs=[
        pl.BlockSpec((1, N, MAX_NB), lambda b: (b, 0, 0)),
        pl.BlockSpec((1, N, MAX_NB), lambda b: (b, 0, 0)),
        pl.BlockSpec((1, N, 1), lambda b: (b, 0, 0)),
    ],
    out_specs=[
        pl.BlockSpec((1, N, MAX_NB), lambda b: (b, 0, 0)),
        pl.BlockSpec((1, N, MAX_NB), lambda b: (b, 0, 0)),
    ],
    out_shape=[
        jax.ShapeDtypeStruct((B, N, MAX_NB), jnp.int32),
        jax.ShapeDtypeStruct((B, N, MAX_NB), jnp.int32),
    ],
)


def _tc1_body(ia_ref, ib_ref, wa_ref, wu2a_ref, wu2b_ref, wnb_ref, bu2_ref,
              af_ref, qa_ref, qb_ref, pb_ref):
    i = pl.program_id(0)
    valid = _valid_rows(i, (128, H))
    af = jnp.dot(ia_ref[...], wa_ref[...], preferred_element_type=_f32)
    af_ref[...] = af
    qa = jnp.dot(af, wu2a_ref[...], preferred_element_type=_f32) + bu2_ref[...]
    qa_ref[...] = jnp.where(valid, qa, NEG)
    ib = ib_ref[...]
    qb = jnp.dot(ib, wu2b_ref[...], preferred_element_type=_f32)
    qb_ref[...] = jnp.where(valid, qb, NEG)
    pb_ref[...] = jnp.dot(ib, wnb_ref[...], preferred_element_type=_f32)


_tc1 = pl.pallas_call(
    _tc1_body,
    grid=(NR // 128,),
    in_specs=[
        pl.BlockSpec((128, ATOM_FDIM), lambda i: (i, 0)),
        pl.BlockSpec((128, BOND_FDIM), lambda i: (i, 0)),
        pl.BlockSpec((ATOM_FDIM, H), lambda i: (0, 0)),
        pl.BlockSpec((H, H), lambda i: (0, 0)),
        pl.BlockSpec((BOND_FDIM, H), lambda i: (0, 0)),
        pl.BlockSpec((BOND_FDIM, H), lambda i: (0, 0)),
        pl.BlockSpec((1, H), lambda i: (0, 0)),
    ],
    out_specs=[pl.BlockSpec((128, H), lambda i: (i, 0))] * 4,
    out_shape=[jax.ShapeDtypeStruct((NR, H), _f32)] * 4,
)


def _upd1_body(af_ref, nl_ref, u1a_ref, u1b_ref, b1_ref, u2a_ref, b2_ref,
               afn_ref, qan_ref):
    i = pl.program_id(0)
    afn = jnp.dot(af_ref[...], u1a_ref[...], preferred_element_type=_f32)
    afn = afn + jnp.dot(nl_ref[...], u1b_ref[...], preferred_element_type=_f32)
    afn = jnp.maximum(afn + b1_ref[...], 0.0)
    afn_ref[...] = afn
    qa = jnp.dot(afn, u2a_ref[...], preferred_element_type=_f32) + b2_ref[...]
    qan_ref[...] = jnp.where(_valid_rows(i, (128, H)), qa, NEG)


_upd1 = pl.pallas_call(
    _upd1_body,
    grid=(NR // 128,),
    in_specs=[
        pl.BlockSpec((128, H), lambda i: (i, 0)),
        pl.BlockSpec((128, H), lambda i: (i, 0)),
        pl.BlockSpec((H, H), lambda i: (0, 0)),
        pl.BlockSpec((H, H), lambda i: (0, 0)),
        pl.BlockSpec((1, H), lambda i: (0, 0)),
        pl.BlockSpec((H, H), lambda i: (0, 0)),
        pl.BlockSpec((1, H), lambda i: (0, 0)),
    ],
    out_specs=[pl.BlockSpec((128, H), lambda i: (i, 0))] * 2,
    out_shape=[jax.ShapeDtypeStruct((NR, H), _f32)] * 2,
)


def _upd2_body(af_ref, nl_ref, u1a_ref, u1b_ref, b1_ref, wna_ref, ws_ref,
               pa_ref, fs_ref):
    afn = jnp.dot(af_ref[...], u1a_ref[...], preferred_element_type=_f32)
    afn = afn + jnp.dot(nl_ref[...], u1b_ref[...], preferred_element_type=_f32)
    afn = jnp.maximum(afn + b1_ref[...], 0.0)
    pa_ref[...] = jnp.dot(afn, wna_ref[...], preferred_element_type=_f32)
    fs_ref[...] = jnp.dot(afn, ws_ref[...], preferred_element_type=_f32)


_upd2 = pl.pallas_call(
    _upd2_body,
    grid=(NR // 128,),
    in_specs=[
        pl.BlockSpec((128, H), lambda i: (i, 0)),
        pl.BlockSpec((128, H), lambda i: (i, 0)),
        pl.BlockSpec((H, H), lambda i: (0, 0)),
        pl.BlockSpec((H, H), lambda i: (0, 0)),
        pl.BlockSpec((1, H), lambda i: (0, 0)),
        pl.BlockSpec((H, H), lambda i: (0, 0)),
        pl.BlockSpec((H, H), lambda i: (0, 0)),
    ],
    out_specs=[pl.BlockSpec((128, H), lambda i: (i, 0))] * 2,
    out_shape=[jax.ShapeDtypeStruct((NR, H), _f32)] * 2,
)


def _red_body(fn_ref, fs_ref, nm_ref, wout_ref, bout_ref, out_ref):
    k = fn_ref[...] * fs_ref[...] * nm_ref[0]
    v = jnp.sum(k, axis=0, keepdims=True)
    r = jnp.dot(v, wout_ref[...], preferred_element_type=_f32) + bout_ref[...]
    out_ref[...] = r.reshape(1, 1, 1)


_reduce = pl.pallas_call(
    _red_body,
    grid=(B,),
    in_specs=[
        pl.BlockSpec((N, H), lambda b: (b, 0)),
        pl.BlockSpec((N, H), lambda b: (b, 0)),
        pl.BlockSpec((1, N, 1), lambda b: (b, 0, 0)),
        pl.BlockSpec((H, 1), lambda b: (0, 0)),
        pl.BlockSpec((1, 1), lambda b: (0, 0)),
    ],
    out_specs=pl.BlockSpec((1, 1, 1), lambda b: (b, 0, 0)),
    out_shape=jax.ShapeDtypeStruct((B, 1, 1), _f32),
)


_SC_MESH = plsc.VectorSubcoreMesh(core_axis_name="c", subcore_axis_name="s")


def _worker_id():
    return lax.axis_index("s") * 2 + lax.axis_index("c")


def _relu_sum_rows(buf, obuf, nrows):
    @plsc.parallel_loop(0, nrows, step=1, unroll=2)
    def _row(r):
        e0 = r * MAX_NB
        for v in range(H // 16):
            sl = pl.ds(v * 16, 16)
            acc = jnp.maximum(buf[e0, sl], 0.0)
            for k in range(1, MAX_NB):
                acc = acc + jnp.maximum(buf[e0 + k, sl], 0.0)
            obuf[r, sl] = acc


def _prod_sum_rows(bufa, bufb, obuf, nrows):
    @plsc.parallel_loop(0, nrows, step=1, unroll=2)
    def _row(r):
        e0 = r * MAX_NB
        for v in range(H // 16):
            sl = pl.ds(v * 16, 16)
            acc = bufa[e0, sl] * bufb[e0, sl]
            for k in range(1, MAX_NB):
                acc = acc + bufa[e0 + k, sl] * bufb[e0 + k, sl]
            obuf[r, sl] = acc


def _sc_add_body(ag_ref, bg_ref, qa_ref, qb_ref, nl_ref,
                 idxa, idxb, buf0, buf1, obuf, semb, sema):
    wid = _worker_id()
    pltpu.sync_copy(ag_ref.at[pl.ds(wid * N_CH * NSUB, N_CH * NSUB)], idxa)
    pltpu.sync_copy(bg_ref.at[pl.ds(wid * N_CH * NSUB, N_CH * NSUB)], idxb)

    def gather_base(ci, buf):
        return [pltpu.async_copy(qb_ref.at[idxb.at[ci * NSUB + j]],
                                 buf.at[pl.ds(j * SUB, SUB)], semb)
                for j in range(NSUB)]

    def gather_add(ci, buf):
        return [pltpu.async_copy(qa_ref.at[idxa.at[ci * NSUB + j]],
                                 buf.at[pl.ds(j * SUB, SUB)], sema, add=True)
                for j in range(NSUB)]

    def finish(ci, buf):
        _relu_sum_rows(buf, obuf, CH_ROWS)
        g = wid * N_CH + ci
        pltpu.sync_copy(obuf, nl_ref.at[pl.ds(g * CH_ROWS, CH_ROWS)])

    def pair_body(cc, _):
        c0 = cc * 2
        c1 = c0 + 1
        with jax.named_scope("dma_b0"):
            b = gather_base(c0, buf0)
            for cp in b:
                cp.wait()
        with jax.named_scope("dma_a0"):
            a0 = gather_add(c0, buf0)
            b1 = gather_base(c1, buf1)
            for cp in a0:
                cp.wait()
        with jax.named_scope("compute0"):
            finish(c0, buf0)
        with jax.named_scope("dma_b1"):
            for cp in b1:
                cp.wait()
        with jax.named_scope("dma_a1"):
            a1 = gather_add(c1, buf1)
            for cp in a1:
                cp.wait()
        with jax.named_scope("compute1"):
            finish(c1, buf1)
        return 0

    lax.fori_loop(0, N_CH // 2, pair_body, 0)


_sc_add = pl.kernel(
    _sc_add_body,
    out_type=jax.ShapeDtypeStruct((NR, H), _f32),
    mesh=_SC_MESH,
    scratch_types=[
        pltpu.VMEM((N_CH * NSUB, SUB), jnp.int32),
        pltpu.VMEM((N_CH * NSUB, SUB), jnp.int32),
        pltpu.VMEM((CH_ROWS * MAX_NB, H), _f32),
        pltpu.VMEM((CH_ROWS * MAX_NB, H), _f32),
        pltpu.VMEM((CH_ROWS, H), _f32),
        pltpu.SemaphoreType.DMA,
        pltpu.SemaphoreType.DMA,
    ],
)

M_ROWS = 16
M_NSUB = M_ROWS * MAX_NB // SUB
M_NCH = ROWS_PER_W // M_ROWS


def _sc_mul_body(ag_ref, bg_ref, pa_ref, pb_ref, fn_ref,
                 idxa, idxb, bufa0, bufb0, bufa1, bufb1, obuf, sem0, sem1):
    wid = _worker_id()
    pltpu.sync_copy(ag_ref.at[pl.ds(wid * M_NCH * M_NSUB, M_NCH * M_NSUB)], idxa)
    pltpu.sync_copy(bg_ref.at[pl.ds(wid * M_NCH * M_NSUB, M_NCH * M_NSUB)], idxb)

    def gather(ci, bufa, bufb, sem):
        cps = [pltpu.async_copy(pa_ref.at[idxa.at[ci * M_NSUB + j]],
                                bufa.at[pl.ds(j * SUB, SUB)], sem)
               for j in range(M_NSUB)]
        cps += [pltpu.async_copy(pb_ref.at[idxb.at[ci * M_NSUB + j]],
                                 bufb.at[pl.ds(j * SUB, SUB)], sem)
                for j in range(M_NSUB)]
        return cps

    def finish(ci, bufa, bufb):
        _prod_sum_rows(bufa, bufb, obuf, M_ROWS)
        g = wid * M_NCH + ci
        pltpu.sync_copy(obuf, fn_ref.at[pl.ds(g * M_ROWS, M_ROWS)])

    def pair_body(cc, _):
        c0 = cc * 2
        c1 = c0 + 1
        g0 = gather(c0, bufa0, bufb0, sem0)
        g1 = gather(c1, bufa1, bufb1, sem1)
        for cp in g0:
            cp.wait()
        finish(c0, bufa0, bufb0)
        for cp in g1:
            cp.wait()
        finish(c1, bufa1, bufb1)
        return 0

    lax.fori_loop(0, M_NCH // 2, pair_body, 0)


_sc_mul = pl.kernel(
    _sc_mul_body,
    out_type=jax.ShapeDtypeStruct((NR, H), _f32),
    mesh=_SC_MESH,
    scratch_types=[
        pltpu.VMEM((M_NCH * M_NSUB, SUB), jnp.int32),
        pltpu.VMEM((M_NCH * M_NSUB, SUB), jnp.int32),
        pltpu.VMEM((M_ROWS * MAX_NB, H), _f32),
        pltpu.VMEM((M_ROWS * MAX_NB, H), _f32),
        pltpu.VMEM((M_ROWS * MAX_NB, H), _f32),
        pltpu.VMEM((M_ROWS * MAX_NB, H), _f32),
        pltpu.VMEM((M_ROWS, H), _f32),
        pltpu.SemaphoreType.DMA,
        pltpu.SemaphoreType.DMA,
    ],
)


def kernel(input_atom, input_bond, atom_graph, bond_graph, num_nbs, node_mask,
           W_atom, W_nei_atom, W_nei_bond, W_self, W_U2, b_U2, W_U1, b_U1,
           W_out, b_out):
    ia = input_atom.reshape(B * N, ATOM_FDIM)
    ib = input_bond.reshape(B * NBONDS, BOND_FDIM)
    IA = jnp.pad(ia, ((0, NR - B * N), (0, 0)))
    IB = jnp.pad(ib, ((0, NR - B * NBONDS), (0, 0)))

    ag = atom_graph.astype(jnp.int32)
    bg = bond_graph.astype(jnp.int32)
    nn = num_nbs.astype(jnp.int32)

    W_U2a, W_U2b = W_U2[:H], W_U2[H:]
    W_U1a, W_U1b = W_U1[:H], W_U1[H:]
    b2 = b_U2.reshape(1, H)
    b1 = b_U1.reshape(1, H)

    AGg, BGg = _prep(ag, bg, nn.reshape(B, N, 1))
    pad_edges = (NR - B * N) * MAX_NB
    AG = jnp.concatenate(
        [AGg.reshape(-1), jnp.full((pad_edges,), PADROW, jnp.int32)])
    BG = jnp.concatenate(
        [BGg.reshape(-1), jnp.full((pad_edges,), PADROW, jnp.int32)])
    AG2 = AG.reshape(EROWS, SUB)
    BG2 = BG.reshape(EROWS, SUB)

    AF0, QA0, QB, PB = _tc1(IA, IB, W_atom, W_U2a, W_U2b, W_nei_bond, b2)

    NL0 = _sc_add(AG2, BG2, QA0, QB)
    AF1, QA1 = _upd1(AF0, NL0, W_U1a, W_U1b, b1, W_U2a, b2)
    NL1 = _sc_add(AG2, BG2, QA1, QB)
    PA2, FS2 = _upd2(AF1, NL1, W_U1a, W_U1b, b1, W_nei_atom, W_self)

    FN = _sc_mul(AG2, BG2, PA2, PB)
    out = _reduce(FN[:B * N], FS2[:B * N], node_mask, W_out,
                  b_out.reshape(1, 1))
    return out.reshape(B, 1)


# speedup vs baseline: 12.7568x; 12.7568x over previous
import jax
import jax.numpy as jnp
from jax import lax
from jax.experimental import pallas as pl
from jax.experimental.pallas import tpu as pltpu
from jax.experimental.pallas import tpu_sc as plsc

B, N, NBONDS = 20, 1000, 1000
MAX_NB = 10
H = 128
ATOM_FDIM = 82
BOND_FDIM = 6

BS = 1024
NR = B * BS
NEG = -1e30

NCORE = 2
NSUBC = 16
GB = 2
NGRP = B // (NCORE * GB)
GROWS = GB * BS
TROWS = GROWS // NSUBC
CH_ROWS = 32
N_CH = TROWS // CH_ROWS
SUB = 80
NSUB = CH_ROWS * MAX_NB // SUB
IDXR = TROWS * MAX_NB // SUB
NBLK = NCORE * NGRP * NSUBC

_f32 = jnp.float32


def _valid_rows(i, shape):
    rows = i * shape[0] + lax.broadcasted_iota(jnp.int32, shape, 0)
    return lax.rem(rows, BS) < N


def _prep_body(ag_ref, bg_ref, nn_ref, ago_ref, bgo_ref):
    b = pl.program_id(0)
    off = lax.rem(b, GB) * BS
    k_idx = lax.broadcasted_iota(jnp.int32, (1, N, MAX_NB), 2)
    valid = k_idx < nn_ref[...]
    ago_ref[...] = jnp.where(valid, ag_ref[...] + off, off + N)
    bgo_ref[...] = jnp.where(valid, bg_ref[...] + off, off + N)


_prep = pl.pallas_call(
    _prep_body,
    grid=(B,),
    in_specs=[
        pl.BlockSpec((1, N, MAX_NB), lambda b: (b, 0, 0)),
        pl.BlockSpec((1, N, MAX_NB), lambda b: (b, 0, 0)),
        pl.BlockSpec((1, N, 1), lambda b: (b, 0, 0)),
    ],
    out_specs=[
        pl.BlockSpec((1, N, MAX_NB), lambda b: (b, 0, 0)),
        pl.BlockSpec((1, N, MAX_NB), lambda b: (b, 0, 0)),
    ],
    out_shape=[
        jax.ShapeDtypeStruct((B, N, MAX_NB), jnp.int32),
        jax.ShapeDtypeStruct((B, N, MAX_NB), jnp.int32),
    ],
)


def _tc1_body(ia_ref, ib_ref, wa_ref, wu2a_ref, wu2b_ref, wnb_ref, bu2_ref,
              af_ref, qa_ref, qb_ref, pb_ref):
    i = pl.program_id(0)
    valid = _valid_rows(i, (128, H))
    af = jnp.dot(ia_ref[...], wa_ref[...], preferred_element_type=_f32)
    af_ref[...] = af
    qa = jnp.dot(af, wu2a_ref[...], preferred_element_type=_f32) + bu2_ref[...]
    qa_ref[...] = jnp.where(valid, qa, NEG)
    ib = ib_ref[...]
    qb = jnp.dot(ib, wu2b_ref[...], preferred_element_type=_f32)
    qb_ref[...] = jnp.where(valid, qb, NEG)
    pb_ref[...] = jnp.dot(ib, wnb_ref[...], preferred_element_type=_f32)


_tc1 = pl.pallas_call(
    _tc1_body,
    grid=(NR // 128,),
    in_specs=[
        pl.BlockSpec((128, ATOM_FDIM), lambda i: (i, 0)),
        pl.BlockSpec((128, BOND_FDIM), lambda i: (i, 0)),
        pl.BlockSpec((ATOM_FDIM, H), lambda i: (0, 0)),
        pl.BlockSpec((H, H), lambda i: (0, 0)),
        pl.BlockSpec((BOND_FDIM, H), lambda i: (0, 0)),
        pl.BlockSpec((BOND_FDIM, H), lambda i: (0, 0)),
        pl.BlockSpec((1, H), lambda i: (0, 0)),
    ],
    out_specs=[pl.BlockSpec((128, H), lambda i: (i, 0))] * 4,
    out_shape=[jax.ShapeDtypeStruct((NR, H), _f32)] * 4,
)


def _upd1_body(af_ref, nl_ref, u1a_ref, u1b_ref, b1_ref, u2a_ref, b2_ref,
               afn_ref, qan_ref):
    i = pl.program_id(0)
    afn = jnp.dot(af_ref[...], u1a_ref[...], preferred_element_type=_f32)
    afn = afn + jnp.dot(nl_ref[...], u1b_ref[...], preferred_element_type=_f32)
    afn = jnp.maximum(afn + b1_ref[...], 0.0)
    afn_ref[...] = afn
    qa = jnp.dot(afn, u2a_ref[...], preferred_element_type=_f32) + b2_ref[...]
    qan_ref[...] = jnp.where(_valid_rows(i, (128, H)), qa, NEG)


_upd1 = pl.pallas_call(
    _upd1_body,
    grid=(NR // 128,),
    in_specs=[
        pl.BlockSpec((128, H), lambda i: (i, 0)),
        pl.BlockSpec((128, H), lambda i: (i, 0)),
        pl.BlockSpec((H, H), lambda i: (0, 0)),
        pl.BlockSpec((H, H), lambda i: (0, 0)),
        pl.BlockSpec((1, H), lambda i: (0, 0)),
        pl.BlockSpec((H, H), lambda i: (0, 0)),
        pl.BlockSpec((1, H), lambda i: (0, 0)),
    ],
    out_specs=[pl.BlockSpec((128, H), lambda i: (i, 0))] * 2,
    out_shape=[jax.ShapeDtypeStruct((NR, H), _f32)] * 2,
)


def _upd2_body(af_ref, nl_ref, u1a_ref, u1b_ref, b1_ref, wna_ref, ws_ref,
               pa_ref, fs_ref):
    afn = jnp.dot(af_ref[...], u1a_ref[...], preferred_element_type=_f32)
    afn = afn + jnp.dot(nl_ref[...], u1b_ref[...], preferred_element_type=_f32)
    afn = jnp.maximum(afn + b1_ref[...], 0.0)
    pa_ref[...] = jnp.dot(afn, wna_ref[...], preferred_element_type=_f32)
    fs_ref[...] = jnp.dot(afn, ws_ref[...], preferred_element_type=_f32)


_upd2 = pl.pallas_call(
    _upd2_body,
    grid=(NR // 128,),
    in_specs=[
        pl.BlockSpec((128, H), lambda i: (i, 0)),
        pl.BlockSpec((128, H), lambda i: (i, 0)),
        pl.BlockSpec((H, H), lambda i: (0, 0)),
        pl.BlockSpec((H, H), lambda i: (0, 0)),
        pl.BlockSpec((1, H), lambda i: (0, 0)),
        pl.BlockSpec((H, H), lambda i: (0, 0)),
        pl.BlockSpec((H, H), lambda i: (0, 0)),
    ],
    out_specs=[pl.BlockSpec((128, H), lambda i: (i, 0))] * 2,
    out_shape=[jax.ShapeDtypeStruct((NR, H), _f32)] * 2,
)


def _red_body(fn_ref, fs_ref, nm_ref, wout_ref, bout_ref, out_ref):
    k = fn_ref[...] * fs_ref[...] * nm_ref[0]
    v = jnp.sum(k, axis=0, keepdims=True)
    r = jnp.dot(v, wout_ref[...], preferred_element_type=_f32) + bout_ref[...]
    out_ref[...] = r.reshape(1, 1, 1)


_reduce = pl.pallas_call(
    _red_body,
    grid=(B,),
    in_specs=[
        pl.BlockSpec((BS, H), lambda b: (b, 0)),
        pl.BlockSpec((BS, H), lambda b: (b, 0)),
        pl.BlockSpec((1, BS, 1), lambda b: (b, 0, 0)),
        pl.BlockSpec((H, 1), lambda b: (0, 0)),
        pl.BlockSpec((1, 1), lambda b: (0, 0)),
    ],
    out_specs=pl.BlockSpec((1, 1, 1), lambda b: (b, 0, 0)),
    out_shape=jax.ShapeDtypeStruct((B, 1, 1), _f32),
)


_SC_MESH = plsc.VectorSubcoreMesh(core_axis_name="c", subcore_axis_name="s")


def _relu_sum_rows(buf, obuf, nrows):
    @plsc.parallel_loop(0, nrows, step=1, unroll=2)
    def _row(r):
        e0 = r * MAX_NB
        for v in range(H // 16):
            sl = pl.ds(v * 16, 16)
            acc = jnp.maximum(buf[e0, sl], 0.0)
            for k in range(1, MAX_NB):
                acc = acc + jnp.maximum(buf[e0 + k, sl], 0.0)
            obuf[r, sl] = acc


def _prod_sum_rows(bufa, bufb, obuf, nrows):
    @plsc.parallel_loop(0, nrows, step=1, unroll=2)
    def _row(r):
        e0 = r * MAX_NB
        for v in range(H // 16):
            sl = pl.ds(v * 16, 16)
            acc = bufa[e0, sl] * bufb[e0, sl]
            for k in range(1, MAX_NB):
                acc = acc + bufa[e0 + k, sl] * bufb[e0 + k, sl]
            obuf[r, sl] = acc


def _sc_add_body(ag_ref, bg_ref, qa_ref, qb_ref, nl_ref,
                 idxa, idxb, buf0, buf1, obuf, qa_s, qb_s, semb, sema):
    c = lax.axis_index("c")
    s = lax.axis_index("s")

    for g in range(NGRP):
        r0 = (c * (NGRP * GB) + g * GB) * BS
        plsc.subcore_barrier()
        @pl.when(s == 0)
        def _stage():
            pltpu.sync_copy(qa_ref.at[pl.ds(r0, GROWS)], qa_s)
            pltpu.sync_copy(qb_ref.at[pl.ds(r0, GROWS)], qb_s)
        plsc.subcore_barrier()

        blk = (c * NGRP + g) * NSUBC + s
        pltpu.sync_copy(ag_ref.at[blk], idxa)
        pltpu.sync_copy(bg_ref.at[blk], idxb)

        def gather_base(ci, buf):
            return [pltpu.async_copy(qb_s.at[idxb.at[ci * NSUB + j]],
                                     buf.at[pl.ds(j * SUB, SUB)], semb)
                    for j in range(NSUB)]

        def gather_add(ci, buf):
            return [pltpu.async_copy(qa_s.at[idxa.at[ci * NSUB + j]],
                                     buf.at[pl.ds(j * SUB, SUB)], sema,
                                     add=True)
                    for j in range(NSUB)]

        def finish(ci, buf):
            _relu_sum_rows(buf, obuf, CH_ROWS)
            row0 = r0 + s * TROWS + ci * CH_ROWS
            pltpu.sync_copy(obuf, nl_ref.at[pl.ds(row0, CH_ROWS)])

        def pair_body(cc, _):
            c0 = cc * 2
            c1 = c0 + 1
            b = gather_base(c0, buf0)
            for cp in b:
                cp.wait()
            a0 = gather_add(c0, buf0)
            b1 = gather_base(c1, buf1)
            for cp in a0:
                cp.wait()
            finish(c0, buf0)
            for cp in b1:
                cp.wait()
            a1 = gather_add(c1, buf1)
            for cp in a1:
                cp.wait()
            finish(c1, buf1)
            return 0

        lax.fori_loop(0, N_CH // 2, pair_body, 0)
    plsc.subcore_barrier()


_sc_add = pl.kernel(
    _sc_add_body,
    out_type=jax.ShapeDtypeStruct((NR, H), _f32),
    mesh=_SC_MESH,
    scratch_types=[
        pltpu.VMEM((IDXR, SUB), jnp.int32),
        pltpu.VMEM((IDXR, SUB), jnp.int32),
        pltpu.VMEM((CH_ROWS * MAX_NB, H), _f32),
        pltpu.VMEM((CH_ROWS * MAX_NB, H), _f32),
        pltpu.VMEM((CH_ROWS, H), _f32),
        pltpu.MemorySpace.VMEM_SHARED((GROWS, H), _f32),
        pltpu.MemorySpace.VMEM_SHARED((GROWS, H), _f32),
        pltpu.SemaphoreType.DMA,
        pltpu.SemaphoreType.DMA,
    ],
)

M_ROWS = 16
M_NSUB = M_ROWS * MAX_NB // SUB
M_NCH = TROWS // M_ROWS


def _sc_mul_body(ag_ref, bg_ref, pa_ref, pb_ref, fn_ref,
                 idxa, idxb, bufa0, bufb0, bufa1, bufb1, obuf, pa_s, pb_s,
                 sem0, sem1):
    c = lax.axis_index("c")
    s = lax.axis_index("s")

    for g in range(NGRP):
        r0 = (c * (NGRP * GB) + g * GB) * BS
        plsc.subcore_barrier()
        @pl.when(s == 0)
        def _stage():
            pltpu.sync_copy(pa_ref.at[pl.ds(r0, GROWS)], pa_s)
            pltpu.sync_copy(pb_ref.at[pl.ds(r0, GROWS)], pb_s)
        plsc.subcore_barrier()

        blk = (c * NGRP + g) * NSUBC + s
        pltpu.sync_copy(ag_ref.at[blk], idxa)
        pltpu.sync_copy(bg_ref.at[blk], idxb)

        def gather(ci, bufa, bufb, sem):
            cps = [pltpu.async_copy(pa_s.at[idxa.at[ci * M_NSUB + j]],
                                    bufa.at[pl.ds(j * SUB, SUB)], sem)
                   for j in range(M_NSUB)]
            cps += [pltpu.async_copy(pb_s.at[idxb.at[ci * M_NSUB + j]],
                                     bufb.at[pl.ds(j * SUB, SUB)], sem)
                    for j in range(M_NSUB)]
            return cps

        def finish(ci, bufa, bufb):
            _prod_sum_rows(bufa, bufb, obuf, M_ROWS)
            row0 = r0 + s * TROWS + ci * M_ROWS
            pltpu.sync_copy(obuf, fn_ref.at[pl.ds(row0, M_ROWS)])

        def pair_body(cc, _):
            c0 = cc * 2
            c1 = c0 + 1
            g0 = gather(c0, bufa0, bufb0, sem0)
            g1 = gather(c1, bufa1, bufb1, sem1)
            for cp in g0:
                cp.wait()
            finish(c0, bufa0, bufb0)
            for cp in g1:
                cp.wait()
            finish(c1, bufa1, bufb1)
            return 0

        lax.fori_loop(0, M_NCH // 2, pair_body, 0)
    plsc.subcore_barrier()


_sc_mul = pl.kernel(
    _sc_mul_body,
    out_type=jax.ShapeDtypeStruct((NR, H), _f32),
    mesh=_SC_MESH,
    scratch_types=[
        pltpu.VMEM((IDXR, SUB), jnp.int32),
        pltpu.VMEM((IDXR, SUB), jnp.int32),
        pltpu.VMEM((M_ROWS * MAX_NB, H), _f32),
        pltpu.VMEM((M_ROWS * MAX_NB, H), _f32),
        pltpu.VMEM((M_ROWS * MAX_NB, H), _f32),
        pltpu.VMEM((M_ROWS * MAX_NB, H), _f32),
        pltpu.VMEM((M_ROWS, H), _f32),
        pltpu.MemorySpace.VMEM_SHARED((GROWS, H), _f32),
        pltpu.MemorySpace.VMEM_SHARED((GROWS, H), _f32),
        pltpu.SemaphoreType.DMA,
        pltpu.SemaphoreType.DMA,
    ],
)


def kernel(input_atom, input_bond, atom_graph, bond_graph, num_nbs, node_mask,
           W_atom, W_nei_atom, W_nei_bond, W_self, W_U2, b_U2, W_U1, b_U1,
           W_out, b_out):
    IA = jnp.pad(input_atom, ((0, 0), (0, BS - N), (0, 0))).reshape(NR, ATOM_FDIM)
    IB = jnp.pad(input_bond, ((0, 0), (0, BS - NBONDS), (0, 0))).reshape(NR, BOND_FDIM)

    ag = atom_graph.astype(jnp.int32)
    bg = bond_graph.astype(jnp.int32)
    nn = num_nbs.astype(jnp.int32)

    W_U2a, W_U2b = W_U2[:H], W_U2[H:]
    W_U1a, W_U1b = W_U1[:H], W_U1[H:]
    b2 = b_U2.reshape(1, H)
    b1 = b_U1.reshape(1, H)

    AGg, BGg = _prep(ag, bg, nn.reshape(B, N, 1))
    padrow = ((jnp.arange(B, dtype=jnp.int32) % GB) * BS + N)[:, None, None]
    padblk = jnp.broadcast_to(padrow, (B, BS - N, MAX_NB))
    AG2 = jnp.concatenate([AGg, padblk], axis=1).reshape(NBLK, IDXR, SUB)
    BG2 = jnp.concatenate([BGg, padblk], axis=1).reshape(NBLK, IDXR, SUB)

    AF0, QA0, QB, PB = _tc1(IA, IB, W_atom, W_U2a, W_U2b, W_nei_bond, b2)

    NL0 = _sc_add(AG2, BG2, QA0, QB)
    AF1, QA1 = _upd1(AF0, NL0, W_U1a, W_U1b, b1, W_U2a, b2)
    NL1 = _sc_add(AG2, BG2, QA1, QB)
    PA2, FS2 = _upd2(AF1, NL1, W_U1a, W_U1b, b1, W_nei_atom, W_self)

    FN = _sc_mul(AG2, BG2, PA2, PB)
    nm = jnp.pad(node_mask, ((0, 0), (0, BS - N), (0, 0)))
    out = _reduce(FN, FS2, nm, W_out, b_out.reshape(1, 1))
    return out.reshape(B, 1)


# speedup vs baseline: 12.8505x; 1.0073x over previous
import jax
import jax.numpy as jnp
from jax import lax
from jax.experimental import pallas as pl
from jax.experimental.pallas import tpu as pltpu
from jax.experimental.pallas import tpu_sc as plsc

B, N, NBONDS = 20, 1000, 1000
MAX_NB = 10
H = 128
ATOM_FDIM = 82
BOND_FDIM = 6

BS = 1024
NR = B * BS
NEG = -1e30

NCORE = 2
NSUBC = 16
GB = 2
NGRP = B // (NCORE * GB)
GROWS = GB * BS
TROWS = GROWS // NSUBC
CH_ROWS = 32
N_CH = TROWS // CH_ROWS
SUB = 80
NSUB = CH_ROWS * MAX_NB // SUB
IDXR = TROWS * MAX_NB // SUB
NBLK = NCORE * NGRP * NSUBC

_f32 = jnp.float32


def _valid_rows(i, shape):
    rows = i * shape[0] + lax.broadcasted_iota(jnp.int32, shape, 0)
    return lax.rem(rows, BS) < N


def _prep_body(ag_ref, bg_ref, nn_ref, ago_ref, bgo_ref):
    b = pl.program_id(0)
    off = lax.rem(b, GB) * BS
    k_idx = lax.broadcasted_iota(jnp.int32, (1, N, MAX_NB), 2)
    valid = k_idx < nn_ref[...]
    ago_ref[...] = jnp.where(valid, ag_ref[...] + off, off + N)
    bgo_ref[...] = jnp.where(valid, bg_ref[...] + off, off + N)


_prep = pl.pallas_call(
    _prep_body,
    grid=(B,),
    in_specs=[
        pl.BlockSpec((1, N, MAX_NB), lambda b: (b, 0, 0)),
        pl.BlockSpec((1, N, MAX_NB), lambda b: (b, 0, 0)),
        pl.BlockSpec((1, N, 1), lambda b: (b, 0, 0)),
    ],
    out_specs=[
        pl.BlockSpec((1, N, MAX_NB), lambda b: (b, 0, 0)),
        pl.BlockSpec((1, N, MAX_NB), lambda b: (b, 0, 0)),
    ],
    out_shape=[
        jax.ShapeDtypeStruct((B, N, MAX_NB), jnp.int32),
        jax.ShapeDtypeStruct((B, N, MAX_NB), jnp.int32),
    ],
)


def _tc1_body(ia_ref, ib_ref, wa_ref, wu2a_ref, wu2b_ref, wnb_ref, bu2_ref,
              af_ref, qa_ref, qb_ref, pb_ref):
    i = pl.program_id(0)
    valid = _valid_rows(i, (128, H))
    af = jnp.dot(ia_ref[...], wa_ref[...], preferred_element_type=_f32)
    af_ref[...] = af
    qa = jnp.dot(af, wu2a_ref[...], preferred_element_type=_f32) + bu2_ref[...]
    qa_ref[...] = jnp.where(valid, qa, NEG)
    ib = ib_ref[...]
    qb = jnp.dot(ib, wu2b_ref[...], preferred_element_type=_f32)
    qb_ref[...] = jnp.where(valid, qb, NEG)
    pb_ref[...] = jnp.dot(ib, wnb_ref[...], preferred_element_type=_f32)


_tc1 = pl.pallas_call(
    _tc1_body,
    grid=(NR // 128,),
    in_specs=[
        pl.BlockSpec((128, ATOM_FDIM), lambda i: (i, 0)),
        pl.BlockSpec((128, BOND_FDIM), lambda i: (i, 0)),
        pl.BlockSpec((ATOM_FDIM, H), lambda i: (0, 0)),
        pl.BlockSpec((H, H), lambda i: (0, 0)),
        pl.BlockSpec((BOND_FDIM, H), lambda i: (0, 0)),
        pl.BlockSpec((BOND_FDIM, H), lambda i: (0, 0)),
        pl.BlockSpec((1, H), lambda i: (0, 0)),
    ],
    out_specs=[pl.BlockSpec((128, H), lambda i: (i, 0))] * 4,
    out_shape=[jax.ShapeDtypeStruct((NR, H), _f32)] * 4,
)


def _upd1_body(af_ref, nl_ref, u1a_ref, u1b_ref, b1_ref, u2a_ref, b2_ref,
               afn_ref, qan_ref):
    i = pl.program_id(0)
    afn = jnp.dot(af_ref[...], u1a_ref[...], preferred_element_type=_f32)
    afn = afn + jnp.dot(nl_ref[...], u1b_ref[...], preferred_element_type=_f32)
    afn = jnp.maximum(afn + b1_ref[...], 0.0)
    afn_ref[...] = afn
    qa = jnp.dot(afn, u2a_ref[...], preferred_element_type=_f32) + b2_ref[...]
    qan_ref[...] = jnp.where(_valid_rows(i, (128, H)), qa, NEG)


_upd1 = pl.pallas_call(
    _upd1_body,
    grid=(NR // 128,),
    in_specs=[
        pl.BlockSpec((128, H), lambda i: (i, 0)),
        pl.BlockSpec((128, H), lambda i: (i, 0)),
        pl.BlockSpec((H, H), lambda i: (0, 0)),
        pl.BlockSpec((H, H), lambda i: (0, 0)),
        pl.BlockSpec((1, H), lambda i: (0, 0)),
        pl.BlockSpec((H, H), lambda i: (0, 0)),
        pl.BlockSpec((1, H), lambda i: (0, 0)),
    ],
    out_specs=[pl.BlockSpec((128, H), lambda i: (i, 0))] * 2,
    out_shape=[jax.ShapeDtypeStruct((NR, H), _f32)] * 2,
)


def _upd2_body(af_ref, nl_ref, u1a_ref, u1b_ref, b1_ref, wna_ref, ws_ref,
               pa_ref, fs_ref):
    afn = jnp.dot(af_ref[...], u1a_ref[...], preferred_element_type=_f32)
    afn = afn + jnp.dot(nl_ref[...], u1b_ref[...], preferred_element_type=_f32)
    afn = jnp.maximum(afn + b1_ref[...], 0.0)
    pa_ref[...] = jnp.dot(afn, wna_ref[...], preferred_element_type=_f32)
    fs_ref[...] = jnp.dot(afn, ws_ref[...], preferred_element_type=_f32)


_upd2 = pl.pallas_call(
    _upd2_body,
    grid=(NR // 128,),
    in_specs=[
        pl.BlockSpec((128, H), lambda i: (i, 0)),
        pl.BlockSpec((128, H), lambda i: (i, 0)),
        pl.BlockSpec((H, H), lambda i: (0, 0)),
        pl.BlockSpec((H, H), lambda i: (0, 0)),
        pl.BlockSpec((1, H), lambda i: (0, 0)),
        pl.BlockSpec((H, H), lambda i: (0, 0)),
        pl.BlockSpec((H, H), lambda i: (0, 0)),
    ],
    out_specs=[pl.BlockSpec((128, H), lambda i: (i, 0))] * 2,
    out_shape=[jax.ShapeDtypeStruct((NR, H), _f32)] * 2,
)


def _red_body(fn_ref, fs_ref, nm_ref, wout_ref, bout_ref, out_ref):
    k = fn_ref[...] * fs_ref[...] * nm_ref[0]
    v = jnp.sum(k, axis=0, keepdims=True)
    r = jnp.dot(v, wout_ref[...], preferred_element_type=_f32) + bout_ref[...]
    out_ref[...] = r.reshape(1, 1, 1)


_reduce = pl.pallas_call(
    _red_body,
    grid=(B,),
    in_specs=[
        pl.BlockSpec((BS, H), lambda b: (b, 0)),
        pl.BlockSpec((BS, H), lambda b: (b, 0)),
        pl.BlockSpec((1, BS, 1), lambda b: (b, 0, 0)),
        pl.BlockSpec((H, 1), lambda b: (0, 0)),
        pl.BlockSpec((1, 1), lambda b: (0, 0)),
    ],
    out_specs=pl.BlockSpec((1, 1, 1), lambda b: (b, 0, 0)),
    out_shape=jax.ShapeDtypeStruct((B, 1, 1), _f32),
)


_SC_MESH = plsc.VectorSubcoreMesh(core_axis_name="c", subcore_axis_name="s")


def _relu_sum_rows(buf, obuf, nrows):
    @plsc.parallel_loop(0, nrows, step=1, unroll=2)
    def _row(r):
        e0 = r * MAX_NB
        for v in range(H // 16):
            sl = pl.ds(v * 16, 16)
            acc = jnp.maximum(buf[e0, sl], 0.0)
            for k in range(1, MAX_NB):
                acc = acc + jnp.maximum(buf[e0 + k, sl], 0.0)
            obuf[r, sl] = acc


def _prod_sum_rows(bufa, bufb, obuf, nrows):
    @plsc.parallel_loop(0, nrows, step=1, unroll=2)
    def _row(r):
        e0 = r * MAX_NB
        for v in range(H // 16):
            sl = pl.ds(v * 16, 16)
            acc = bufa[e0, sl] * bufb[e0, sl]
            for k in range(1, MAX_NB):
                acc = acc + bufa[e0 + k, sl] * bufb[e0 + k, sl]
            obuf[r, sl] = acc


def _sc_add_body(ag_ref, bg_ref, qa_ref, qb_ref, nl_ref,
                 idxa, idxb, buf0, buf1, obuf, qa_s, qb_s, semb, sema):
    c = lax.axis_index("c")
    s = lax.axis_index("s")

    for g in range(NGRP):
        r0 = (c * (NGRP * GB) + g * GB) * BS
        plsc.subcore_barrier()
        srow = s * (GROWS // NSUBC)
        pltpu.sync_copy(qa_ref.at[pl.ds(r0 + srow, GROWS // NSUBC)],
                        qa_s.at[pl.ds(srow, GROWS // NSUBC)])
        pltpu.sync_copy(qb_ref.at[pl.ds(r0 + srow, GROWS // NSUBC)],
                        qb_s.at[pl.ds(srow, GROWS // NSUBC)])
        plsc.subcore_barrier()

        blk = (c * NGRP + g) * NSUBC + s
        pltpu.sync_copy(ag_ref.at[blk], idxa)
        pltpu.sync_copy(bg_ref.at[blk], idxb)

        def gather_base(ci, buf):
            return [pltpu.async_copy(qb_s.at[idxb.at[ci * NSUB + j]],
                                     buf.at[pl.ds(j * SUB, SUB)], semb)
                    for j in range(NSUB)]

        def gather_add(ci, buf):
            return [pltpu.async_copy(qa_s.at[idxa.at[ci * NSUB + j]],
                                     buf.at[pl.ds(j * SUB, SUB)], sema,
                                     add=True)
                    for j in range(NSUB)]

        def finish(ci, buf):
            _relu_sum_rows(buf, obuf, CH_ROWS)
            row0 = r0 + s * TROWS + ci * CH_ROWS
            pltpu.sync_copy(obuf, nl_ref.at[pl.ds(row0, CH_ROWS)])

        def pair_body(cc, _):
            c0 = cc * 2
            c1 = c0 + 1
            b = gather_base(c0, buf0)
            for cp in b:
                cp.wait()
            a0 = gather_add(c0, buf0)
            b1 = gather_base(c1, buf1)
            for cp in a0:
                cp.wait()
            finish(c0, buf0)
            for cp in b1:
                cp.wait()
            a1 = gather_add(c1, buf1)
            for cp in a1:
                cp.wait()
            finish(c1, buf1)
            return 0

        lax.fori_loop(0, N_CH // 2, pair_body, 0)
    plsc.subcore_barrier()


_sc_add = pl.kernel(
    _sc_add_body,
    out_type=jax.ShapeDtypeStruct((NR, H), _f32),
    mesh=_SC_MESH,
    scratch_types=[
        pltpu.VMEM((IDXR, SUB), jnp.int32),
        pltpu.VMEM((IDXR, SUB), jnp.int32),
        pltpu.VMEM((CH_ROWS * MAX_NB, H), _f32),
        pltpu.VMEM((CH_ROWS * MAX_NB, H), _f32),
        pltpu.VMEM((CH_ROWS, H), _f32),
        pltpu.MemorySpace.VMEM_SHARED((GROWS, H), _f32),
        pltpu.MemorySpace.VMEM_SHARED((GROWS, H), _f32),
        pltpu.SemaphoreType.DMA,
        pltpu.SemaphoreType.DMA,
    ],
)

M_ROWS = 16
M_NSUB = M_ROWS * MAX_NB // SUB
M_NCH = TROWS // M_ROWS


def _sc_mul_body(ag_ref, bg_ref, pa_ref, pb_ref, fn_ref,
                 idxa, idxb, bufa0, bufb0, bufa1, bufb1, obuf, pa_s, pb_s,
                 sem0, sem1):
    c = lax.axis_index("c")
    s = lax.axis_index("s")

    for g in range(NGRP):
        r0 = (c * (NGRP * GB) + g * GB) * BS
        plsc.subcore_barrier()
        srow = s * (GROWS // NSUBC)
        pltpu.sync_copy(pa_ref.at[pl.ds(r0 + srow, GROWS // NSUBC)],
                        pa_s.at[pl.ds(srow, GROWS // NSUBC)])
        pltpu.sync_copy(pb_ref.at[pl.ds(r0 + srow, GROWS // NSUBC)],
                        pb_s.at[pl.ds(srow, GROWS // NSUBC)])
        plsc.subcore_barrier()

        blk = (c * NGRP + g) * NSUBC + s
        pltpu.sync_copy(ag_ref.at[blk], idxa)
        pltpu.sync_copy(bg_ref.at[blk], idxb)

        def gather(ci, bufa, bufb, sem):
            cps = [pltpu.async_copy(pa_s.at[idxa.at[ci * M_NSUB + j]],
                                    bufa.at[pl.ds(j * SUB, SUB)], sem)
                   for j in range(M_NSUB)]
            cps += [pltpu.async_copy(pb_s.at[idxb.at[ci * M_NSUB + j]],
                                     bufb.at[pl.ds(j * SUB, SUB)], sem)
                    for j in range(M_NSUB)]
            return cps

        def finish(ci, bufa, bufb):
            _prod_sum_rows(bufa, bufb, obuf, M_ROWS)
            row0 = r0 + s * TROWS + ci * M_ROWS
            pltpu.sync_copy(obuf, fn_ref.at[pl.ds(row0, M_ROWS)])

        def pair_body(cc, _):
            c0 = cc * 2
            c1 = c0 + 1
            g0 = gather(c0, bufa0, bufb0, sem0)
            g1 = gather(c1, bufa1, bufb1, sem1)
            for cp in g0:
                cp.wait()
            finish(c0, bufa0, bufb0)
            for cp in g1:
                cp.wait()
            finish(c1, bufa1, bufb1)
            return 0

        lax.fori_loop(0, M_NCH // 2, pair_body, 0)
    plsc.subcore_barrier()


_sc_mul = pl.kernel(
    _sc_mul_body,
    out_type=jax.ShapeDtypeStruct((NR, H), _f32),
    mesh=_SC_MESH,
    scratch_types=[
        pltpu.VMEM((IDXR, SUB), jnp.int32),
        pltpu.VMEM((IDXR, SUB), jnp.int32),
        pltpu.VMEM((M_ROWS * MAX_NB, H), _f32),
        pltpu.VMEM((M_ROWS * MAX_NB, H), _f32),
        pltpu.VMEM((M_ROWS * MAX_NB, H), _f32),
        pltpu.VMEM((M_ROWS * MAX_NB, H), _f32),
        pltpu.VMEM((M_ROWS, H), _f32),
        pltpu.MemorySpace.VMEM_SHARED((GROWS, H), _f32),
        pltpu.MemorySpace.VMEM_SHARED((GROWS, H), _f32),
        pltpu.SemaphoreType.DMA,
        pltpu.SemaphoreType.DMA,
    ],
)


def kernel(input_atom, input_bond, atom_graph, bond_graph, num_nbs, node_mask,
           W_atom, W_nei_atom, W_nei_bond, W_self, W_U2, b_U2, W_U1, b_U1,
           W_out, b_out):
    IA = jnp.pad(input_atom, ((0, 0), (0, BS - N), (0, 0))).reshape(NR, ATOM_FDIM)
    IB = jnp.pad(input_bond, ((0, 0), (0, BS - NBONDS), (0, 0))).reshape(NR, BOND_FDIM)

    ag = atom_graph.astype(jnp.int32)
    bg = bond_graph.astype(jnp.int32)
    nn = num_nbs.astype(jnp.int32)

    W_U2a, W_U2b = W_U2[:H], W_U2[H:]
    W_U1a, W_U1b = W_U1[:H], W_U1[H:]
    b2 = b_U2.reshape(1, H)
    b1 = b_U1.reshape(1, H)

    AGg, BGg = _prep(ag, bg, nn.reshape(B, N, 1))
    padrow = ((jnp.arange(B, dtype=jnp.int32) % GB) * BS + N)[:, None, None]
    padblk = jnp.broadcast_to(padrow, (B, BS - N, MAX_NB))
    AG2 = jnp.concatenate([AGg, padblk], axis=1).reshape(NBLK, IDXR, SUB)
    BG2 = jnp.concatenate([BGg, padblk], axis=1).reshape(NBLK, IDXR, SUB)

    AF0, QA0, QB, PB = _tc1(IA, IB, W_atom, W_U2a, W_U2b, W_nei_bond, b2)

    NL0 = _sc_add(AG2, BG2, QA0, QB)
    AF1, QA1 = _upd1(AF0, NL0, W_U1a, W_U1b, b1, W_U2a, b2)
    NL1 = _sc_add(AG2, BG2, QA1, QB)
    PA2, FS2 = _upd2(AF1, NL1, W_U1a, W_U1b, b1, W_nei_atom, W_self)

    FN = _sc_mul(AG2, BG2, PA2, PB)
    nm = jnp.pad(node_mask, ((0, 0), (0, BS - N), (0, 0)))
    out = _reduce(FN, FS2, nm, W_out, b_out.reshape(1, 1))
    return out.reshape(B, 1)


# speedup vs baseline: 17.6244x; 1.3715x over previous
import jax
import jax.numpy as jnp
from jax import lax
from jax.experimental import pallas as pl
from jax.experimental.pallas import tpu as pltpu
from jax.experimental.pallas import tpu_sc as plsc

B, N, NBONDS = 20, 1000, 1000
MAX_NB = 10
H = 128
ATOM_FDIM = 82
BOND_FDIM = 6

BS = 1024
NR = B * BS
NEG = -1e30

NCORE = 2
NSUBC = 16
GB = 2
NGRP = B // (NCORE * GB)
GROWS = GB * BS
TROWS = GROWS // NSUBC
CH_ROWS = 32
N_CH = TROWS // CH_ROWS
SUB = 80
NSUB = CH_ROWS * MAX_NB // SUB
IDXR = TROWS * MAX_NB // SUB
NBLK = NCORE * NGRP * NSUBC

_f32 = jnp.float32
RB = 2048


def _valid_rows(i, shape):
    rows = i * shape[0] + lax.broadcasted_iota(jnp.int32, shape, 0)
    return lax.rem(rows, BS) < N


def _prep_body(ag_ref, bg_ref, nn_ref, ago_ref, bgo_ref):
    b = pl.program_id(0)
    off = lax.rem(b, GB) * BS
    k_idx = lax.broadcasted_iota(jnp.int32, (1, N, MAX_NB), 2)
    valid = k_idx < nn_ref[...]
    ago_ref[...] = jnp.where(valid, ag_ref[...] + off, off + N)
    bgo_ref[...] = jnp.where(valid, bg_ref[...] + off, off + N)


_prep = pl.pallas_call(
    _prep_body,
    grid=(B,),
    in_specs=[
        pl.BlockSpec((1, N, MAX_NB), lambda b: (b, 0, 0)),
        pl.BlockSpec((1, N, MAX_NB), lambda b: (b, 0, 0)),
        pl.BlockSpec((1, N, 1), lambda b: (b, 0, 0)),
    ],
    out_specs=[
        pl.BlockSpec((1, N, MAX_NB), lambda b: (b, 0, 0)),
        pl.BlockSpec((1, N, MAX_NB), lambda b: (b, 0, 0)),
    ],
    out_shape=[
        jax.ShapeDtypeStruct((B, N, MAX_NB), jnp.int32),
        jax.ShapeDtypeStruct((B, N, MAX_NB), jnp.int32),
    ],
)


def _tc1_body(ia_ref, ib_ref, wa_ref, wu2a_ref, wu2b_ref, wnb_ref, bu2_ref,
              af_ref, qa_ref, qb_ref, pb_ref):
    i = pl.program_id(0)
    valid = _valid_rows(i, (RB, H))
    af = jnp.dot(ia_ref[...], wa_ref[...], preferred_element_type=_f32)
    af_ref[...] = af
    qa = jnp.dot(af, wu2a_ref[...], preferred_element_type=_f32) + bu2_ref[...]
    qa_ref[...] = jnp.where(valid, qa, NEG)
    ib = ib_ref[...]
    qb = jnp.dot(ib, wu2b_ref[...], preferred_element_type=_f32)
    qb_ref[...] = jnp.where(valid, qb, NEG)
    pb_ref[...] = jnp.dot(ib, wnb_ref[...], preferred_element_type=_f32)


_tc1 = pl.pallas_call(
    _tc1_body,
    grid=(NR // RB,),
    in_specs=[
        pl.BlockSpec((RB, ATOM_FDIM), lambda i: (i, 0)),
        pl.BlockSpec((RB, BOND_FDIM), lambda i: (i, 0)),
        pl.BlockSpec((ATOM_FDIM, H), lambda i: (0, 0)),
        pl.BlockSpec((H, H), lambda i: (0, 0)),
        pl.BlockSpec((BOND_FDIM, H), lambda i: (0, 0)),
        pl.BlockSpec((BOND_FDIM, H), lambda i: (0, 0)),
        pl.BlockSpec((1, H), lambda i: (0, 0)),
    ],
    out_specs=[pl.BlockSpec((RB, H), lambda i: (i, 0))] * 4,
    out_shape=[jax.ShapeDtypeStruct((NR, H), _f32)] * 4,
)


def _upd1_body(af_ref, nl_ref, u1a_ref, u1b_ref, b1_ref, u2a_ref, b2_ref,
               afn_ref, qan_ref):
    i = pl.program_id(0)
    afn = jnp.dot(af_ref[...], u1a_ref[...], preferred_element_type=_f32)
    afn = afn + jnp.dot(nl_ref[...], u1b_ref[...], preferred_element_type=_f32)
    afn = jnp.maximum(afn + b1_ref[...], 0.0)
    afn_ref[...] = afn
    qa = jnp.dot(afn, u2a_ref[...], preferred_element_type=_f32) + b2_ref[...]
    qan_ref[...] = jnp.where(_valid_rows(i, (RB, H)), qa, NEG)


_upd1 = pl.pallas_call(
    _upd1_body,
    grid=(NR // RB,),
    in_specs=[
        pl.BlockSpec((RB, H), lambda i: (i, 0)),
        pl.BlockSpec((RB, H), lambda i: (i, 0)),
        pl.BlockSpec((H, H), lambda i: (0, 0)),
        pl.BlockSpec((H, H), lambda i: (0, 0)),
        pl.BlockSpec((1, H), lambda i: (0, 0)),
        pl.BlockSpec((H, H), lambda i: (0, 0)),
        pl.BlockSpec((1, H), lambda i: (0, 0)),
    ],
    out_specs=[pl.BlockSpec((RB, H), lambda i: (i, 0))] * 2,
    out_shape=[jax.ShapeDtypeStruct((NR, H), _f32)] * 2,
)


def _upd2_body(af_ref, nl_ref, u1a_ref, u1b_ref, b1_ref, wna_ref, ws_ref,
               pa_ref, fs_ref):
    afn = jnp.dot(af_ref[...], u1a_ref[...], preferred_element_type=_f32)
    afn = afn + jnp.dot(nl_ref[...], u1b_ref[...], preferred_element_type=_f32)
    afn = jnp.maximum(afn + b1_ref[...], 0.0)
    pa_ref[...] = jnp.dot(afn, wna_ref[...], preferred_element_type=_f32)
    fs_ref[...] = jnp.dot(afn, ws_ref[...], preferred_element_type=_f32)


_upd2 = pl.pallas_call(
    _upd2_body,
    grid=(NR // RB,),
    in_specs=[
        pl.BlockSpec((RB, H), lambda i: (i, 0)),
        pl.BlockSpec((RB, H), lambda i: (i, 0)),
        pl.BlockSpec((H, H), lambda i: (0, 0)),
        pl.BlockSpec((H, H), lambda i: (0, 0)),
        pl.BlockSpec((1, H), lambda i: (0, 0)),
        pl.BlockSpec((H, H), lambda i: (0, 0)),
        pl.BlockSpec((H, H), lambda i: (0, 0)),
    ],
    out_specs=[pl.BlockSpec((RB, H), lambda i: (i, 0))] * 2,
    out_shape=[jax.ShapeDtypeStruct((NR, H), _f32)] * 2,
)


RED_B = 4


def _red_body(fn_ref, fs_ref, nm_ref, wout_ref, bout_ref, out_ref):
    k = fn_ref[...] * fs_ref[...] * nm_ref[...].reshape(RED_B * BS, 1)
    v = jnp.sum(k.reshape(RED_B, BS, H), axis=1)
    r = jnp.dot(v, wout_ref[...], preferred_element_type=_f32) + bout_ref[...]
    out_ref[...] = r.reshape(RED_B, 1, 1)


_reduce = pl.pallas_call(
    _red_body,
    grid=(B // RED_B,),
    in_specs=[
        pl.BlockSpec((RED_B * BS, H), lambda b: (b, 0)),
        pl.BlockSpec((RED_B * BS, H), lambda b: (b, 0)),
        pl.BlockSpec((RED_B, BS, 1), lambda b: (b, 0, 0)),
        pl.BlockSpec((H, 1), lambda b: (0, 0)),
        pl.BlockSpec((1, 1), lambda b: (0, 0)),
    ],
    out_specs=pl.BlockSpec((RED_B, 1, 1), lambda b: (b, 0, 0)),
    out_shape=jax.ShapeDtypeStruct((B, 1, 1), _f32),
)


_SC_MESH = plsc.VectorSubcoreMesh(core_axis_name="c", subcore_axis_name="s")


def _relu_sum_rows(buf, obuf, nrows):
    @plsc.parallel_loop(0, nrows, step=1, unroll=2)
    def _row(r):
        e0 = r * MAX_NB
        for v in range(H // 16):
            sl = pl.ds(v * 16, 16)
            acc = jnp.maximum(buf[e0, sl], 0.0)
            for k in range(1, MAX_NB):
                acc = acc + jnp.maximum(buf[e0 + k, sl], 0.0)
            obuf[r, sl] = acc


def _prod_sum_rows(bufa, bufb, obuf, nrows):
    @plsc.parallel_loop(0, nrows, step=1, unroll=2)
    def _row(r):
        e0 = r * MAX_NB
        for v in range(H // 16):
            sl = pl.ds(v * 16, 16)
            acc = bufa[e0, sl] * bufb[e0, sl]
            for k in range(1, MAX_NB):
                acc = acc + bufa[e0 + k, sl] * bufb[e0 + k, sl]
            obuf[r, sl] = acc


def _sc_add_body(ag_ref, bg_ref, qa_ref, qb_ref, nl_ref,
                 idxa, idxb, buf0, buf1, obuf, qa_s, qb_s, semb, sema):
    c = lax.axis_index("c")
    s = lax.axis_index("s")

    for g in range(NGRP):
        r0 = (c * (NGRP * GB) + g * GB) * BS
        plsc.subcore_barrier()
        srow = s * (GROWS // NSUBC)
        pltpu.sync_copy(qa_ref.at[pl.ds(r0 + srow, GROWS // NSUBC)],
                        qa_s.at[pl.ds(srow, GROWS // NSUBC)])
        pltpu.sync_copy(qb_ref.at[pl.ds(r0 + srow, GROWS // NSUBC)],
                        qb_s.at[pl.ds(srow, GROWS // NSUBC)])
        plsc.subcore_barrier()

        blk = (c * NGRP + g) * NSUBC + s
        pltpu.sync_copy(ag_ref.at[blk], idxa)
        pltpu.sync_copy(bg_ref.at[blk], idxb)

        def gather_base(ci, buf):
            return [pltpu.async_copy(qb_s.at[idxb.at[ci * NSUB + j]],
                                     buf.at[pl.ds(j * SUB, SUB)], semb)
                    for j in range(NSUB)]

        def gather_add(ci, buf):
            return [pltpu.async_copy(qa_s.at[idxa.at[ci * NSUB + j]],
                                     buf.at[pl.ds(j * SUB, SUB)], sema,
                                     add=True)
                    for j in range(NSUB)]

        def finish(ci, buf):
            _relu_sum_rows(buf, obuf, CH_ROWS)
            row0 = r0 + s * TROWS + ci * CH_ROWS
            pltpu.sync_copy(obuf, nl_ref.at[pl.ds(row0, CH_ROWS)])

        def pair_body(cc, _):
            c0 = cc * 2
            c1 = c0 + 1
            b = gather_base(c0, buf0)
            for cp in b:
                cp.wait()
            a0 = gather_add(c0, buf0)
            b1 = gather_base(c1, buf1)
            for cp in a0:
                cp.wait()
            finish(c0, buf0)
            for cp in b1:
                cp.wait()
            a1 = gather_add(c1, buf1)
            for cp in a1:
                cp.wait()
            finish(c1, buf1)
            return 0

        lax.fori_loop(0, N_CH // 2, pair_body, 0)
    plsc.subcore_barrier()


_sc_add = pl.kernel(
    _sc_add_body,
    out_type=jax.ShapeDtypeStruct((NR, H), _f32),
    mesh=_SC_MESH,
    scratch_types=[
        pltpu.VMEM((IDXR, SUB), jnp.int32),
        pltpu.VMEM((IDXR, SUB), jnp.int32),
        pltpu.VMEM((CH_ROWS * MAX_NB, H), _f32),
        pltpu.VMEM((CH_ROWS * MAX_NB, H), _f32),
        pltpu.VMEM((CH_ROWS, H), _f32),
        pltpu.MemorySpace.VMEM_SHARED((GROWS, H), _f32),
        pltpu.MemorySpace.VMEM_SHARED((GROWS, H), _f32),
        pltpu.SemaphoreType.DMA,
        pltpu.SemaphoreType.DMA,
    ],
)

M_ROWS = 16
M_NSUB = M_ROWS * MAX_NB // SUB
M_NCH = TROWS // M_ROWS


def _sc_mul_body(ag_ref, bg_ref, pa_ref, pb_ref, fn_ref,
                 idxa, idxb, bufa0, bufb0, bufa1, bufb1, obuf, pa_s, pb_s,
                 sem0, sem1):
    c = lax.axis_index("c")
    s = lax.axis_index("s")

    for g in range(NGRP):
        r0 = (c * (NGRP * GB) + g * GB) * BS
        plsc.subcore_barrier()
        srow = s * (GROWS // NSUBC)
        pltpu.sync_copy(pa_ref.at[pl.ds(r0 + srow, GROWS // NSUBC)],
                        pa_s.at[pl.ds(srow, GROWS // NSUBC)])
        pltpu.sync_copy(pb_ref.at[pl.ds(r0 + srow, GROWS // NSUBC)],
                        pb_s.at[pl.ds(srow, GROWS // NSUBC)])
        plsc.subcore_barrier()

        blk = (c * NGRP + g) * NSUBC + s
        pltpu.sync_copy(ag_ref.at[blk], idxa)
        pltpu.sync_copy(bg_ref.at[blk], idxb)

        def gather(ci, bufa, bufb, sem):
            cps = [pltpu.async_copy(pa_s.at[idxa.at[ci * M_NSUB + j]],
                                    bufa.at[pl.ds(j * SUB, SUB)], sem)
                   for j in range(M_NSUB)]
            cps += [pltpu.async_copy(pb_s.at[idxb.at[ci * M_NSUB + j]],
                                     bufb.at[pl.ds(j * SUB, SUB)], sem)
                    for j in range(M_NSUB)]
            return cps

        def finish(ci, bufa, bufb):
            _prod_sum_rows(bufa, bufb, obuf, M_ROWS)
            row0 = r0 + s * TROWS + ci * M_ROWS
            pltpu.sync_copy(obuf, fn_ref.at[pl.ds(row0, M_ROWS)])

        def pair_body(cc, _):
            c0 = cc * 2
            c1 = c0 + 1
            g0 = gather(c0, bufa0, bufb0, sem0)
            g1 = gather(c1, bufa1, bufb1, sem1)
            for cp in g0:
                cp.wait()
            finish(c0, bufa0, bufb0)
            for cp in g1:
                cp.wait()
            finish(c1, bufa1, bufb1)
            return 0

        lax.fori_loop(0, M_NCH // 2, pair_body, 0)
    plsc.subcore_barrier()


_sc_mul = pl.kernel(
    _sc_mul_body,
    out_type=jax.ShapeDtypeStruct((NR, H), _f32),
    mesh=_SC_MESH,
    scratch_types=[
        pltpu.VMEM((IDXR, SUB), jnp.int32),
        pltpu.VMEM((IDXR, SUB), jnp.int32),
        pltpu.VMEM((M_ROWS * MAX_NB, H), _f32),
        pltpu.VMEM((M_ROWS * MAX_NB, H), _f32),
        pltpu.VMEM((M_ROWS * MAX_NB, H), _f32),
        pltpu.VMEM((M_ROWS * MAX_NB, H), _f32),
        pltpu.VMEM((M_ROWS, H), _f32),
        pltpu.MemorySpace.VMEM_SHARED((GROWS, H), _f32),
        pltpu.MemorySpace.VMEM_SHARED((GROWS, H), _f32),
        pltpu.SemaphoreType.DMA,
        pltpu.SemaphoreType.DMA,
    ],
)


def kernel(input_atom, input_bond, atom_graph, bond_graph, num_nbs, node_mask,
           W_atom, W_nei_atom, W_nei_bond, W_self, W_U2, b_U2, W_U1, b_U1,
           W_out, b_out):
    IA = jnp.pad(input_atom, ((0, 0), (0, BS - N), (0, 0))).reshape(NR, ATOM_FDIM)
    IB = jnp.pad(input_bond, ((0, 0), (0, BS - NBONDS), (0, 0))).reshape(NR, BOND_FDIM)

    ag = atom_graph.astype(jnp.int32)
    bg = bond_graph.astype(jnp.int32)
    nn = num_nbs.astype(jnp.int32)

    W_U2a, W_U2b = W_U2[:H], W_U2[H:]
    W_U1a, W_U1b = W_U1[:H], W_U1[H:]
    b2 = b_U2.reshape(1, H)
    b1 = b_U1.reshape(1, H)

    AGg, BGg = _prep(ag, bg, nn.reshape(B, N, 1))
    padrow = ((jnp.arange(B, dtype=jnp.int32) % GB) * BS + N)[:, None, None]
    padblk = jnp.broadcast_to(padrow, (B, BS - N, MAX_NB))
    AG2 = jnp.concatenate([AGg, padblk], axis=1).reshape(NBLK, IDXR, SUB)
    BG2 = jnp.concatenate([BGg, padblk], axis=1).reshape(NBLK, IDXR, SUB)

    AF0, QA0, QB, PB = _tc1(IA, IB, W_atom, W_U2a, W_U2b, W_nei_bond, b2)

    NL0 = _sc_add(AG2, BG2, QA0, QB)
    AF1, QA1 = _upd1(AF0, NL0, W_U1a, W_U1b, b1, W_U2a, b2)
    NL1 = _sc_add(AG2, BG2, QA1, QB)
    PA2, FS2 = _upd2(AF1, NL1, W_U1a, W_U1b, b1, W_nei_atom, W_self)

    FN = _sc_mul(AG2, BG2, PA2, PB)
    nm = jnp.pad(node_mask, ((0, 0), (0, BS - N), (0, 0)))
    out = _reduce(FN, FS2, nm, W_out, b_out.reshape(1, 1))
    return out.reshape(B, 1)
